# Initial kernel scaffold; baseline (speedup 1.0000x reference)
import jax
import jax.numpy as jnp
from jax.experimental import pallas as pl


def kernel(coords, features_3d, query_points):
    raise NotImplementedError("write your pallas kernel here")


# speedup vs baseline: 8.0866x; 8.0866x over previous
import jax
import jax.numpy as jnp
from jax import lax
from jax.experimental import pallas as pl
from jax.experimental.pallas import tpu as pltpu
from jax.experimental.pallas import tpu_sc as plsc

GRID = 128
G3 = GRID * GRID * GRID
IGNORE = -1
NC, NS, L = 2, 16, 16
NW = NC * NS

SLAB = G3 // NW
CHUNK_A = 6272

CQ = 1024
NCORNER = 8
CQ8 = CQ * NCORNER

CORNERS = [(dx, dy, dz) for dx in (0, 1) for dy in (0, 1) for dz in (0, 1)]


def _occ_build(cells_pad):
    vp = cells_pad.shape[0]
    assert vp % CHUNK_A == 0
    mesh = plsc.VectorSubcoreMesh(core_axis_name="c", subcore_axis_name="s")

    def body(cells_hbm, occ_hbm, slab_ref, buf_ref):
        c = lax.axis_index("c")
        s = lax.axis_index("s")
        w = c * NS + s
        lo = w * SLAB
        neg1 = jnp.full((L,), IGNORE, jnp.int32)

        def init(i, carry):
            slab_ref[pl.ds(i * L, L)] = neg1
            return carry

        lax.fori_loop(0, SLAB // L, init, 0)

        def chunk(ci, carry):
            pltpu.sync_copy(cells_hbm.at[pl.ds(ci * CHUNK_A, CHUNK_A)], buf_ref)

            def grp(g, carry2):
                cell = buf_ref[pl.ds(g * L, L)]
                rel = cell - lo
                mask = (rel >= 0) & (rel < SLAB)
                vid = (ci * CHUNK_A + g * L) + lax.iota(jnp.int32, L)
                plsc.store_scatter(
                    slab_ref, [jnp.where(mask, rel, 0)], vid, mask=mask)
                return carry2

            lax.fori_loop(0, CHUNK_A // L, grp, 0)
            return carry

        lax.fori_loop(0, vp // CHUNK_A, chunk, 0)
        pltpu.sync_copy(slab_ref, occ_hbm.at[pl.ds(lo, SLAB)])

    f = pl.kernel(
        body,
        out_type=jax.ShapeDtypeStruct((G3,), jnp.int32),
        mesh=mesh,
        compiler_params=pltpu.CompilerParams(needs_layout_passes=False, use_tc_tiling_on_sc=False),
        scratch_types=[
            pltpu.VMEM((SLAB,), jnp.int32),
            pltpu.VMEM((CHUNK_A,), jnp.int32),
        ],
    )
    return f(cells_pad)


def _main(lqx, lqy, lqz, occ, features, nvox):
    qp = lqx.shape[0]
    fdim = features.shape[1]
    assert qp % CQ == 0 and fdim == 32
    nchunk = qp // CQ
    iters = (nchunk + NW - 1) // NW
    vsl = 6272
    mesh = plsc.VectorSubcoreMesh(core_axis_name="c", subcore_axis_name="s")

    def body(qx_hbm, qy_hbm, qz_hbm, occ_hbm, feat_hbm,
             qf_hbm, oidx_hbm, ow_hbm, part_hbm,
             qx_ref, qy_ref, qz_ref,
             cells_ref, ovals_ref, wraw_ref, inr_ref,
             oidx_ref, ow_ref,
             cmp_idx_ref, cmp_w_ref, cmp_q_ref,
             qf_ref, rows_ref, accum_sp, zeros_ref, sem):
        c = lax.axis_index("c")
        s = lax.axis_index("s")
        w = c * NS + s

        def z(i, carry):
            zeros_ref[pl.ds(i * L, L)] = jnp.zeros((L,), jnp.float32)
            return carry
        lax.fori_loop(0, vsl // L, z, 0)
        astart = jnp.minimum(s * vsl, nvox - vsl)
        pltpu.sync_copy(zeros_ref, accum_sp.at[pl.ds(astart, vsl)])
        plsc.subcore_barrier()

        def do_chunk(cid):
            qbase = cid * CQ
            pltpu.sync_copy(qx_hbm.at[pl.ds(qbase, CQ)], qx_ref)
            pltpu.sync_copy(qy_hbm.at[pl.ds(qbase, CQ)], qy_ref)
            pltpu.sync_copy(qz_hbm.at[pl.ds(qbase, CQ)], qz_ref)

            def p1(g, carry):
                x = qx_ref[pl.ds(g * L, L)]
                y = qy_ref[pl.ds(g * L, L)]
                z_ = qz_ref[pl.ds(g * L, L)]

                def fl(v):
                    t = v.astype(jnp.int32)
                    t = t - jnp.where(t.astype(jnp.float32) > v,
                                      jnp.int32(1), jnp.int32(0))
                    return t, v - t.astype(jnp.float32)

                bx, fx = fl(x)
                by, fy = fl(y)
                bz, fz = fl(z_)
                wx = (1.0 - fx, fx)
                wy = (1.0 - fy, fy)
                wz = (1.0 - fz, fz)
                for k, (dx, dy, dz) in enumerate(CORNERS):
                    cx = bx + dx
                    cy = by + dy
                    cz = bz + dz
                    inr = ((cx >= 0) & (cx < GRID) & (cy >= 0) & (cy < GRID)
                           & (cz >= 0) & (cz < GRID))
                    ccx = jnp.minimum(jnp.maximum(cx, 0), GRID - 1)
                    ccy = jnp.minimum(jnp.maximum(cy, 0), GRID - 1)
                    ccz = jnp.minimum(jnp.maximum(cz, 0), GRID - 1)
                    cell = (ccx * GRID + ccy) * GRID + ccz
                    wk = jnp.where(inr, (wx[dx] * wy[dy]) * wz[dz], 0.0)
                    p16 = k * (CQ // L) + g
                    row = p16 // 8
                    col = (g % 8) * L
                    cells_ref[row, pl.ds(col, L)] = cell
                    wraw_ref[pl.ds(p16 * L, L)] = wk
                    inr_ref[pl.ds(p16 * L, L)] = jnp.where(
                        inr, jnp.int32(1), jnp.int32(0))
                return carry

            lax.fori_loop(0, CQ // L, p1, 0)

            descs = []
            for j in range(CQ8 // 128):
                descs.append(pltpu.async_copy(
                    occ_hbm.at[cells_ref.at[j]], ovals_ref.at[j], sem))
            for d in descs:
                d.wait()

            def zq(i, carry):
                qf_ref[pl.ds(i * L, L)] = jnp.zeros((L,), jnp.float32)
                return carry
            lax.fori_loop(0, CQ * fdim // L, zq, 0)

            def p2(p, cnt):
                row = p // 8
                col = (p % 8) * L
                ov = ovals_ref[row, pl.ds(col, L)]
                wr = wraw_ref[pl.ds(p * L, L)]
                ir = inr_ref[pl.ds(p * L, L)]
                occd = ov >= 0
                valid = occd & (ir != 0)
                wfin = jnp.where(occd, wr, 0.0)
                iout = jnp.where(valid, ov, IGNORE)
                k = p // (CQ // L)
                ql = (p % (CQ // L)) * L + lax.iota(jnp.int32, L)
                tgt = ql * NCORNER + k
                plsc.store_scatter(oidx_ref, [tgt], iout)
                plsc.store_scatter(ow_ref, [tgt], wfin)
                plsc.store_compressed(cmp_idx_ref.at[pl.ds(cnt, L)],
                                      jnp.where(valid, ov, 0), mask=valid)
                plsc.store_compressed(cmp_w_ref.at[pl.ds(cnt, L)],
                                      wfin, mask=valid)
                plsc.store_compressed(cmp_q_ref.at[pl.ds(cnt, L)],
                                      ql, mask=valid)
                return cnt + jnp.sum(valid.astype(jnp.int32))

            cnt = lax.fori_loop(0, CQ8 // L, p2, jnp.int32(0))

            zi = jnp.zeros((L,), jnp.int32)
            zf = jnp.zeros((L,), jnp.float32)
            for j in range(128 // L):
                cmp_idx_ref[pl.ds(cnt + j * L, L)] = zi
                cmp_w_ref[pl.ds(cnt + j * L, L)] = zf
                cmp_q_ref[pl.ds(cnt + j * L, L)] = zi

            def batch(b, carry):
                bs = b * 128
                pltpu.async_copy(
                    feat_hbm.at[cmp_idx_ref.at[pl.ds(bs, 128)]],
                    rows_ref, sem).wait()
                for g8 in range(128 // L):
                    qv = cmp_q_ref[pl.ds(bs + g8 * L, L)]
                    wv = cmp_w_ref[pl.ds(bs + g8 * L, L)]
                    for e in range(L):
                        qe = qv[e]
                        we = wv[e]
                        r0 = rows_ref[g8 * L + e, pl.ds(0, L)]
                        r1 = rows_ref[g8 * L + e, pl.ds(L, L)]
                        a0 = qf_ref[pl.ds(qe * fdim, L)]
                        a1 = qf_ref[pl.ds(qe * fdim + L, L)]
                        qf_ref[pl.ds(qe * fdim, L)] = a0 + we * r0
                        qf_ref[pl.ds(qe * fdim + L, L)] = a1 + we * r1
                return carry

            nb = (cnt + 127) // 128
            lax.fori_loop(0, nb, batch, 0)

            def acc(a, carry):
                idxv = cmp_idx_ref[pl.ds(a * L, L)]
                pltpu.sync_copy(cmp_w_ref.at[pl.ds(a * L, L)],
                                accum_sp.at[idxv], add=True)
                return carry

            na = (cnt + (L - 1)) // L
            lax.fori_loop(0, na, acc, 0)

            pltpu.sync_copy(oidx_ref, oidx_hbm.at[pl.ds(qbase * NCORNER, CQ8)])
            pltpu.sync_copy(ow_ref, ow_hbm.at[pl.ds(qbase * NCORNER, CQ8)])
            pltpu.sync_copy(qf_ref, qf_hbm.at[pl.ds(qbase * fdim, CQ * fdim)])

        def loop_i(i, carry):
            cid = w + NW * i

            @pl.when(cid < nchunk)
            def _():
                do_chunk(cid)

            return carry

        lax.fori_loop(0, iters, loop_i, 0)

        plsc.subcore_barrier()
        pltpu.sync_copy(accum_sp.at[pl.ds(astart, vsl)], zeros_ref)
        pltpu.sync_copy(zeros_ref, part_hbm.at[pl.ds(c * nvox + astart, vsl)])

    f = pl.kernel(
        body,
        out_type=(
            jax.ShapeDtypeStruct((qp * fdim,), jnp.float32),
            jax.ShapeDtypeStruct((qp * NCORNER,), jnp.int32),
            jax.ShapeDtypeStruct((qp * NCORNER,), jnp.float32),
            jax.ShapeDtypeStruct((NC * nvox,), jnp.float32),
        ),
        mesh=mesh,
        compiler_params=pltpu.CompilerParams(needs_layout_passes=False, use_tc_tiling_on_sc=False),
        scratch_types=[
            pltpu.VMEM((CQ,), jnp.float32),
            pltpu.VMEM((CQ,), jnp.float32),
            pltpu.VMEM((CQ,), jnp.float32),
            pltpu.VMEM((CQ8 // 128, 128), jnp.int32),
            pltpu.VMEM((CQ8 // 128, 128), jnp.int32),
            pltpu.VMEM((CQ8,), jnp.float32),
            pltpu.VMEM((CQ8,), jnp.int32),
            pltpu.VMEM((CQ8,), jnp.int32),
            pltpu.VMEM((CQ8,), jnp.float32),
            pltpu.VMEM((CQ8 + 144,), jnp.int32),
            pltpu.VMEM((CQ8 + 144,), jnp.float32),
            pltpu.VMEM((CQ8 + 144,), jnp.int32),
            pltpu.VMEM((CQ * 32,), jnp.float32),
            pltpu.VMEM((128, 32), jnp.float32),
            pltpu.VMEM_SHARED((nvox,), jnp.float32),
            pltpu.VMEM((6272,), jnp.float32),
            pltpu.SemaphoreType.DMA,
        ],
    )
    return f(lqx, lqy, lqz, occ, features)


def _sum_parts(parts, nvox):
    mesh = plsc.VectorSubcoreMesh(core_axis_name="c", subcore_axis_name="s")
    vsl = 3136

    def body(part_hbm, accum_hbm, a_ref, b_ref, o_ref):
        c = lax.axis_index("c")
        s = lax.axis_index("s")
        w = c * NS + s
        start = jnp.minimum(w * vsl, nvox - vsl)
        pltpu.sync_copy(part_hbm.at[pl.ds(start, vsl)], a_ref)
        pltpu.sync_copy(part_hbm.at[pl.ds(nvox + start, vsl)], b_ref)

        def add(i, carry):
            o_ref[pl.ds(i * L, L)] = (a_ref[pl.ds(i * L, L)]
                                      + b_ref[pl.ds(i * L, L)])
            return carry

        lax.fori_loop(0, vsl // L, add, 0)
        pltpu.sync_copy(o_ref, accum_hbm.at[pl.ds(start, vsl)])

    f = pl.kernel(
        body,
        out_type=jax.ShapeDtypeStruct((nvox,), jnp.float32),
        mesh=mesh,
        compiler_params=pltpu.CompilerParams(needs_layout_passes=False, use_tc_tiling_on_sc=False),
        scratch_types=[
            pltpu.VMEM((vsl,), jnp.float32),
            pltpu.VMEM((vsl,), jnp.float32),
            pltpu.VMEM((vsl,), jnp.float32),
        ],
    )
    return f(parts)


def kernel(coords, features_3d, query_points):
    nvox, fdim = features_3d.shape
    q = query_points.shape[0]

    shift = jnp.min(coords[:, 1:], axis=0)
    cxyz = coords[:, 1:] - shift
    cells = (cxyz[:, 0] * GRID + cxyz[:, 1]) * GRID + cxyz[:, 2]
    vp = ((nvox + CHUNK_A - 1) // CHUNK_A) * CHUNK_A
    cells_pad = jnp.concatenate(
        [cells, jnp.full((vp - nvox,), -1, jnp.int32)])

    lq = query_points[:, 1:] - shift.astype(query_points.dtype)
    qp = ((q + CQ - 1) // CQ) * CQ
    pad = jnp.full((qp - q,), -4.0, jnp.float32)
    lqx = jnp.concatenate([lq[:, 0], pad])
    lqy = jnp.concatenate([lq[:, 1], pad])
    lqz = jnp.concatenate([lq[:, 2], pad])

    occ = _occ_build(cells_pad)
    qf_flat, oidx_flat, ow_flat, parts = _main(
        lqx, lqy, lqz, occ, features_3d, nvox)
    accum = _sum_parts(parts, nvox)

    qf = qf_flat.reshape(qp, fdim)[:q]
    oidx = oidx_flat.reshape(qp, NCORNER)[:q]
    ow = ow_flat.reshape(qp, NCORNER)[:q]
    return qf, oidx, ow, accum
